# baseline (speedup 1.0000x reference)
import jax
import jax.numpy as jnp
import numpy as np
from jax import lax
from jax.experimental import pallas as pl
from jax.experimental.pallas import tpu as pltpu
from jax.experimental.pallas import tpu_sc as plsc

D_MODEL = 64
SEQ_LEN = 200
N_SEQ = 4096
SCALE = 8.0

NC, NS = 2, 16
NW = NC * NS
ST = N_SEQ // NW
NDT = D_MODEL // 8


def _pos_encoding() -> np.ndarray:
    position = np.arange(0, 512, dtype=np.float64)[:, None]
    div_term = np.exp(
        -np.arange(0, D_MODEL, 2, dtype=np.float64) * (np.log(10000.0) / D_MODEL)
    )
    pe = np.zeros((512, D_MODEL), dtype=np.float32)
    pe[:, 0::2] = np.sin(position * div_term)
    pe[:, 1::2] = np.cos(position * div_term)
    return pe[:SEQ_LEN]


_PES = np.ascontiguousarray(
    np.broadcast_to(_pos_encoding()[:, :, None], (SEQ_LEN, D_MODEL, 16))
)


def _body(idx_hbm, pes_hbm, table_hbm, out_hbm, idx_v, idxT_v, rows_v, pes_v, outT_v, gsem, psem, wsem):
    wid = lax.axis_index("s") * NC + lax.axis_index("c")
    s0 = pl.multiple_of(wid * ST, ST)
    pltpu.sync_copy(idx_hbm.at[pl.ds(s0, ST)], idx_v)
    iota = lax.iota(jnp.int32, 16)

    def tr(t, carry):
        tv = jnp.full((16,), t, jnp.int32)
        for j in range(8):
            idxT_v[t, pl.ds(16 * j, 16)] = plsc.load_gather(
                idx_v, [iota + 16 * j, tv]
            )
        return carry

    lax.fori_loop(0, SEQ_LEN, tr, 0)

    def g_start(t, b):
        pltpu.async_copy(table_hbm.at[idxT_v.at[t]], rows_v.at[b], gsem)

    def g_wait(b):
        pltpu.make_async_copy(table_hbm.at[idxT_v.at[0]], rows_v.at[b], gsem).wait()

    def p_start(t, b):
        pltpu.async_copy(pes_hbm.at[t], pes_v.at[b], psem)

    def p_wait(b):
        pltpu.make_async_copy(pes_hbm.at[0], pes_v.at[b], psem).wait()

    def w_start(t, b):
        pltpu.async_copy(outT_v.at[b], out_hbm.at[t, :, wid], wsem)

    def w_wait(b):
        pltpu.make_async_copy(outT_v.at[b], out_hbm.at[0, :, wid], wsem).wait()

    for b in range(2):
        g_start(b, b)
        p_start(b, b)

    def step(tt, carry):
        for b in range(2):
            t = tt * 2 + b
            g_wait(b)
            p_wait(b)

            @pl.when(tt > 0)
            def _():
                w_wait(b)

            for q in range(NDT):

                def dl(di, carry):
                    d = q * 8 + di
                    peb = pes_v[b, d]
                    dv = jnp.full((16,), d, jnp.int32)
                    bv = jnp.full((16,), b, jnp.int32)
                    for j in range(8):
                        v = plsc.load_gather(rows_v, [bv, iota + 16 * j, dv])
                        outT_v[b, q, di, pl.ds(16 * j, 16)] = v * SCALE + peb
                    return carry

                lax.fori_loop(0, 8, dl, 0)

            w_start(t, b)

            @pl.when(tt < SEQ_LEN // 2 - 1)
            def _():
                g_start(t + 2, b)
                p_start(t + 2, b)
        return carry

    lax.fori_loop(0, SEQ_LEN // 2, step, 0)
    w_wait(0)
    w_wait(1)


def kernel(x, table):
    idx = x.astype(jnp.int32)
    pes = jnp.asarray(_PES)
    call = pl.kernel(
        _body,
        out_type=jax.ShapeDtypeStruct((SEQ_LEN, NDT, NW, 8, 128), jnp.float32),
        mesh=plsc.VectorSubcoreMesh(core_axis_name="c", subcore_axis_name="s"),
        scratch_types=[
            pltpu.VMEM((ST, SEQ_LEN), jnp.int32),
            pltpu.VMEM((SEQ_LEN, ST), jnp.int32),
            pltpu.VMEM((2, ST, D_MODEL), jnp.float32),
            pltpu.VMEM((2, D_MODEL, 16), jnp.float32),
            pltpu.VMEM((2, NDT, 8, 128), jnp.float32),
            pltpu.SemaphoreType.DMA,
            pltpu.SemaphoreType.DMA,
            pltpu.SemaphoreType.DMA,
        ],
        compiler_params=pltpu.CompilerParams(
            use_tc_tiling_on_sc=False, needs_layout_passes=False
        ),
    )
    out5 = call(idx, pes, table)
    return out5.transpose((2, 4, 0, 1, 3)).reshape(N_SEQ, SEQ_LEN, D_MODEL)


# speedup vs baseline: 1.6258x; 1.6258x over previous
import jax
import jax.numpy as jnp
import numpy as np
from jax import lax
from jax.experimental import pallas as pl
from jax.experimental.pallas import tpu as pltpu
from jax.experimental.pallas import tpu_sc as plsc

D_MODEL = 64
SEQ_LEN = 200
N_SEQ = 4096
SCALE = 8.0

NC, NS = 2, 16
NW = NC * NS
ST = N_SEQ // NW
NDT = D_MODEL // 8
NB = 4


def _pos_encoding() -> np.ndarray:
    position = np.arange(0, 512, dtype=np.float64)[:, None]
    div_term = np.exp(
        -np.arange(0, D_MODEL, 2, dtype=np.float64) * (np.log(10000.0) / D_MODEL)
    )
    pe = np.zeros((512, D_MODEL), dtype=np.float32)
    pe[:, 0::2] = np.sin(position * div_term)
    pe[:, 1::2] = np.cos(position * div_term)
    return pe[:SEQ_LEN]


_PES = np.ascontiguousarray(
    np.broadcast_to(_pos_encoding()[:, :, None], (SEQ_LEN, D_MODEL, 16))
)


def _body(idx_hbm, pes_hbm, table_hbm, out_hbm, idx_v, idxT_v, rows_v, pes_v, outT_v, *sems):
    gsems, psems, wsems = sems[0:NB], sems[NB : 2 * NB], sems[2 * NB : 3 * NB]
    wid = lax.axis_index("s") * NC + lax.axis_index("c")
    s0 = pl.multiple_of(wid * ST, ST)
    pltpu.sync_copy(idx_hbm.at[pl.ds(s0, ST)], idx_v)
    iota = lax.iota(jnp.int32, 16)
    rowi = [iota + 16 * j for j in range(8)]

    @plsc.parallel_loop(0, SEQ_LEN)
    def _tr(t):
        tv = jnp.full((16,), t, jnp.int32)
        for j in range(8):
            idxT_v[t, pl.ds(16 * j, 16)] = plsc.load_gather(idx_v, [rowi[j], tv])

    def g_start(t, b):
        pltpu.async_copy(table_hbm.at[idxT_v.at[t]], rows_v.at[b], gsems[b])

    def g_wait(b):
        pltpu.make_async_copy(table_hbm.at[idxT_v.at[0]], rows_v.at[b], gsems[b]).wait()

    def p_start(t, b):
        pltpu.async_copy(pes_hbm.at[t], pes_v.at[b], psems[b])

    def p_wait(b):
        pltpu.make_async_copy(pes_hbm.at[0], pes_v.at[b], psems[b]).wait()

    def w_start(t, b):
        pltpu.async_copy(outT_v.at[b], out_hbm.at[t, :, wid], wsems[b])

    def w_wait(b):
        pltpu.make_async_copy(outT_v.at[b], out_hbm.at[0, :, wid], wsems[b]).wait()

    for t in range(NB - 1):
        g_start(t, t)
        p_start(t, t)

    def step(tt, carry):
        for b in range(NB):
            t = tt * NB + b
            nb = (b + NB - 1) % NB

            @pl.when(t + NB - 1 < SEQ_LEN)
            def _():
                g_start(t + NB - 1, nb)
                p_start(t + NB - 1, nb)

            g_wait(b)
            p_wait(b)

            @pl.when(tt > 0)
            def _():
                w_wait(b)

            @plsc.parallel_loop(0, D_MODEL, unroll=2)
            def _fma(d):
                q = lax.shift_right_logical(d, 3)
                di = lax.bitwise_and(d, 7)
                peb = pes_v[b, d]
                dv = jnp.full((16,), d, jnp.int32)
                for j in range(8):
                    v = plsc.load_gather(rows_v, [jnp.full((16,), b, jnp.int32), rowi[j], dv])
                    outT_v[b, q, di, pl.ds(16 * j, 16)] = v * SCALE + peb

            w_start(t, b)
        return carry

    lax.fori_loop(0, SEQ_LEN // NB, step, 0)
    for b in range(NB):
        w_wait(b)


def kernel(x, table):
    idx = x.astype(jnp.int32)
    pes = jnp.asarray(_PES)
    call = pl.kernel(
        _body,
        out_type=jax.ShapeDtypeStruct((SEQ_LEN, NDT, NW, 8, 128), jnp.float32),
        mesh=plsc.VectorSubcoreMesh(core_axis_name="c", subcore_axis_name="s"),
        scratch_types=[
            pltpu.VMEM((ST, SEQ_LEN), jnp.int32),
            pltpu.VMEM((SEQ_LEN, ST), jnp.int32),
            pltpu.VMEM((NB, ST, D_MODEL), jnp.float32),
            pltpu.VMEM((NB, D_MODEL, 16), jnp.float32),
            pltpu.VMEM((NB, NDT, 8, 128), jnp.float32),
        ]
        + [pltpu.SemaphoreType.DMA] * (3 * NB),
        compiler_params=pltpu.CompilerParams(
            use_tc_tiling_on_sc=False, needs_layout_passes=False
        ),
    )
    out5 = call(idx, pes, table)
    return out5.transpose((2, 4, 0, 1, 3)).reshape(N_SEQ, SEQ_LEN, D_MODEL)
